# Initial kernel scaffold; baseline (speedup 1.0000x reference)
import jax
import jax.numpy as jnp
from jax.experimental import pallas as pl


def kernel(output, y):
    raise NotImplementedError("write your pallas kernel here")


# speedup vs baseline: 1.3710x; 1.3710x over previous
import functools

import jax
import jax.numpy as jnp
from jax import lax
from jax.experimental import pallas as pl
from jax.experimental.pallas import tpu as pltpu
from jax.experimental.pallas import tpu_sc as plsc

_NUM_CORES = 2
_NUM_SUBCORES = 16
_NW = _NUM_CORES * _NUM_SUBCORES
_LANES = 16
_CHUNK = 128


@functools.lru_cache(maxsize=None)
def _make_sc_gather(B: int, C: int):
    b_per_w = B // _NW
    n_chunks = b_per_w // _CHUNK

    mesh = plsc.VectorSubcoreMesh(core_axis_name="c", subcore_axis_name="s")

    @functools.partial(
        pl.kernel,
        mesh=mesh,
        out_type=jax.ShapeDtypeStruct((B,), jnp.float32),
        scratch_types=[
            pltpu.VMEM((b_per_w,), jnp.int32),
            pltpu.VMEM((b_per_w,), jnp.int32),
            pltpu.VMEM((b_per_w,), jnp.float32),
            pltpu.SemaphoreType.DMA,
        ],
    )
    def sc_gather(flat_hbm, y_hbm, out_hbm, y_v, idx_v, val_v, sem):
        wid = lax.axis_index("s") * _NUM_CORES + lax.axis_index("c")
        base = wid * b_per_w
        pltpu.sync_copy(y_hbm.at[pl.ds(base, b_per_w)], y_v)
        for k in range(b_per_w // _LANES):
            rows = base + k * _LANES + lax.iota(jnp.int32, _LANES)
            idx_v[pl.ds(k * _LANES, _LANES)] = (
                rows * C + y_v[pl.ds(k * _LANES, _LANES)]
            )
        copies = []
        for j in range(n_chunks):
            copies.append(
                pltpu.async_copy(
                    flat_hbm.at[idx_v.at[pl.ds(j * _CHUNK, _CHUNK)]],
                    val_v.at[pl.ds(j * _CHUNK, _CHUNK)],
                    sem,
                )
            )
        for c in copies:
            c.wait()
        pltpu.sync_copy(val_v, out_hbm.at[pl.ds(base, b_per_w)])

    return sc_gather


def _tc_hinge_body(C, x_ref, t_ref, o_ref):
    x = x_ref[...]
    t = t_ref[...]
    h = jnp.maximum(x - t + 1.0, 0.0)
    o_ref[...] = (jnp.sum(h, axis=1, keepdims=True) - 1.0) * (1.0 / C)


@functools.lru_cache(maxsize=None)
def _make_tc_hinge(B: int, C: int, block_rows: int):
    return pl.pallas_call(
        functools.partial(_tc_hinge_body, C),
        grid=(B // block_rows,),
        in_specs=[
            pl.BlockSpec((block_rows, C), lambda i: (i, 0)),
            pl.BlockSpec((block_rows, 1), lambda i: (i, 0)),
        ],
        out_specs=pl.BlockSpec((block_rows, 1), lambda i: (i, 0)),
        out_shape=jax.ShapeDtypeStruct((B, 1), jnp.float32),
    )


@jax.jit
def kernel(output, y):
    B, C = output.shape
    y32 = y.astype(jnp.int32)
    t = _make_sc_gather(B, C)(output.reshape(B * C), y32)
    loss = _make_tc_hinge(B, C, 256)(output, t.reshape(B, 1))
    return loss.reshape(B)


# speedup vs baseline: 1.5601x; 1.1379x over previous
import functools

import jax
import jax.numpy as jnp
from jax import lax
from jax.experimental import pallas as pl
from jax.experimental.pallas import tpu as pltpu
from jax.experimental.pallas import tpu_sc as plsc

_NUM_CORES = 2
_NUM_SUBCORES = 16
_NW = _NUM_CORES * _NUM_SUBCORES
_LANES = 16
_CHUNK = 128


@functools.lru_cache(maxsize=None)
def _make_sc_gather(B: int, C: int):
    b_per_w = B // _NW
    n_chunks = b_per_w // _CHUNK

    mesh = plsc.VectorSubcoreMesh(core_axis_name="c", subcore_axis_name="s")

    @functools.partial(
        pl.kernel,
        mesh=mesh,
        out_type=jax.ShapeDtypeStruct((B,), jnp.float32),
        scratch_types=[
            pltpu.VMEM((b_per_w,), jnp.int32),
            pltpu.VMEM((b_per_w,), jnp.int32),
            pltpu.VMEM((b_per_w,), jnp.float32),
            pltpu.SemaphoreType.DMA,
        ],
    )
    def sc_gather(flat_hbm, y_hbm, out_hbm, y_v, idx_v, val_v, sem):
        wid = lax.axis_index("s") * _NUM_CORES + lax.axis_index("c")
        base = wid * b_per_w
        pltpu.sync_copy(y_hbm.at[pl.ds(base, b_per_w)], y_v)
        for k in range(b_per_w // _LANES):
            rows = base + k * _LANES + lax.iota(jnp.int32, _LANES)
            idx_v[pl.ds(k * _LANES, _LANES)] = (
                rows * C + y_v[pl.ds(k * _LANES, _LANES)]
            )
        copies = []
        for j in range(n_chunks):
            copies.append(
                pltpu.async_copy(
                    flat_hbm.at[idx_v.at[pl.ds(j * _CHUNK, _CHUNK)]],
                    val_v.at[pl.ds(j * _CHUNK, _CHUNK)],
                    sem,
                )
            )
        for c in copies:
            c.wait()
        pltpu.sync_copy(val_v, out_hbm.at[pl.ds(base, b_per_w)])

    return sc_gather


def _tc_hinge_body(C, x_ref, t_ref, o_ref):
    x = x_ref[...]
    t = t_ref[...]
    h = jnp.maximum(x - t + 1.0, 0.0)
    o_ref[...] = (jnp.sum(h, axis=1, keepdims=True) - 1.0) * (1.0 / C)


@functools.lru_cache(maxsize=None)
def _make_tc_hinge(B: int, C: int, block_rows: int):
    return pl.pallas_call(
        functools.partial(_tc_hinge_body, C),
        grid=(B // block_rows,),
        in_specs=[
            pl.BlockSpec((block_rows, C), lambda i: (i, 0)),
            pl.BlockSpec((block_rows, 1), lambda i: (i, 0)),
        ],
        out_specs=pl.BlockSpec((block_rows, 1), lambda i: (i, 0)),
        out_shape=jax.ShapeDtypeStruct((B, 1), jnp.float32),
    )


@jax.jit
def kernel(output, y):
    B, C = output.shape
    y32 = y.astype(jnp.int32)
    t = _make_sc_gather(B, C)(output.reshape(B * C), y32)
    loss = _make_tc_hinge(B, C, 1024)(output, t.reshape(B, 1))
    return loss.reshape(B)


# speedup vs baseline: 1.5776x; 1.0112x over previous
import functools

import jax
import jax.numpy as jnp
from jax import lax
from jax.experimental import pallas as pl
from jax.experimental.pallas import tpu as pltpu
from jax.experimental.pallas import tpu_sc as plsc

_NUM_CORES = 2
_NUM_SUBCORES = 16
_NW = _NUM_CORES * _NUM_SUBCORES
_LANES = 16
_CHUNK = 128


@functools.lru_cache(maxsize=None)
def _make_sc_gather(B: int, C: int):
    b_per_w = B // _NW
    n_chunks = b_per_w // _CHUNK

    mesh = plsc.VectorSubcoreMesh(core_axis_name="c", subcore_axis_name="s")

    @functools.partial(
        pl.kernel,
        mesh=mesh,
        out_type=jax.ShapeDtypeStruct((B,), jnp.float32),
        scratch_types=[
            pltpu.VMEM((b_per_w,), jnp.int32),
            pltpu.VMEM((b_per_w,), jnp.int32),
            pltpu.VMEM((b_per_w,), jnp.float32),
            pltpu.SemaphoreType.DMA,
        ],
    )
    def sc_gather(flat_hbm, y_hbm, out_hbm, y_v, idx_v, val_v, sem):
        wid = lax.axis_index("s") * _NUM_CORES + lax.axis_index("c")
        base = wid * b_per_w
        pltpu.sync_copy(y_hbm.at[pl.ds(base, b_per_w)], y_v)
        for k in range(b_per_w // _LANES):
            rows = base + k * _LANES + lax.iota(jnp.int32, _LANES)
            idx_v[pl.ds(k * _LANES, _LANES)] = (
                rows * C + y_v[pl.ds(k * _LANES, _LANES)]
            )
        copies = []
        for j in range(n_chunks):
            copies.append(
                pltpu.async_copy(
                    flat_hbm.at[idx_v.at[pl.ds(j * _CHUNK, _CHUNK)]],
                    val_v.at[pl.ds(j * _CHUNK, _CHUNK)],
                    sem,
                )
            )
        for c in copies:
            c.wait()
        pltpu.sync_copy(val_v, out_hbm.at[pl.ds(base, b_per_w)])

    return sc_gather


def _tc_hinge_body(C, x_ref, t_ref, o_ref):
    x = x_ref[...]
    t = t_ref[...]
    h = jnp.maximum(x - t + 1.0, 0.0)
    o_ref[...] = (jnp.sum(h, axis=1, keepdims=True) - 1.0) * (1.0 / C)


@functools.lru_cache(maxsize=None)
def _make_tc_hinge(B: int, C: int, block_rows: int):
    return pl.pallas_call(
        functools.partial(_tc_hinge_body, C),
        grid=(B // block_rows,),
        in_specs=[
            pl.BlockSpec((block_rows, C), lambda i: (i, 0)),
            pl.BlockSpec((block_rows, 1), lambda i: (i, 0)),
        ],
        out_specs=pl.BlockSpec((block_rows, 1), lambda i: (i, 0)),
        out_shape=jax.ShapeDtypeStruct((B, 1), jnp.float32),
    )


@jax.jit
def kernel(output, y):
    B, C = output.shape
    y32 = y.astype(jnp.int32)
    t = _make_sc_gather(B, C)(output.reshape(B * C), y32)
    loss = _make_tc_hinge(B, C, 2048)(output, t.reshape(B, 1))
    return loss.reshape(B)


# speedup vs baseline: 1.6246x; 1.0298x over previous
import functools

import jax
import jax.numpy as jnp
from jax import lax
from jax.experimental import pallas as pl
from jax.experimental.pallas import tpu as pltpu
from jax.experimental.pallas import tpu_sc as plsc

_NUM_CORES = 2
_NUM_SUBCORES = 16
_NW = _NUM_CORES * _NUM_SUBCORES
_LANES = 16
_CHUNK_ROWS = 32


@functools.lru_cache(maxsize=None)
def _make_sc_hinge(B: int, C: int):
    R = B // _NW
    n_chunks = R // _CHUNK_ROWS
    chunk = _CHUNK_ROWS * C

    mesh = plsc.VectorSubcoreMesh(core_axis_name="c", subcore_axis_name="s")

    @functools.partial(
        pl.kernel,
        mesh=mesh,
        out_type=jax.ShapeDtypeStruct((B,), jnp.float32),
        compiler_params=pltpu.CompilerParams(needs_layout_passes=False),
        scratch_types=[
            pltpu.VMEM((R,), jnp.int32),
            pltpu.VMEM((R,), jnp.float32),
            pltpu.VMEM((chunk,), jnp.float32),
            pltpu.VMEM((chunk,), jnp.float32),
            pltpu.SemaphoreType.DMA,
            pltpu.SemaphoreType.DMA,
        ],
    )
    def sc_hinge(x_hbm, y_hbm, out_hbm, y_v, out_v, xb0, xb1, sem0, sem1):
        wid = lax.axis_index("s") * _NUM_CORES + lax.axis_index("c")
        base = wid * R
        pltpu.sync_copy(y_hbm.at[pl.ds(base, R)], y_v)

        bufs = (xb0, xb1)
        sems = (sem0, sem1)

        def start(ci):
            b = ci % 2
            return pltpu.async_copy(
                x_hbm.at[pl.ds((base + ci * _CHUNK_ROWS) * C, chunk)],
                bufs[b],
                sems[b],
            )

        pending = [start(0), None]
        lanes = lax.iota(jnp.int32, _LANES)
        zero = jnp.zeros((_LANES,), jnp.float32)

        for ci in range(n_chunks):
            if ci + 1 < n_chunks:
                pending[(ci + 1) % 2] = start(ci + 1)
            pending[ci % 2].wait()
            buf = bufs[ci % 2]
            for g in range(_CHUNK_ROWS // _LANES):
                loc = ci * _CHUNK_ROWS + g * _LANES
                row_base = (lanes + g * _LANES) * C
                yv = y_v[pl.ds(loc, _LANES)]
                t = plsc.load_gather(buf, [row_base + yv])
                a = 1.0 - t

                def jbody(i, carry, buf=buf, row_base=row_base, a=a):
                    a0, a1, a2, a3 = carry
                    j = i * 4
                    x0 = plsc.load_gather(buf, [row_base + j])
                    x1 = plsc.load_gather(buf, [row_base + (j + 1)])
                    x2 = plsc.load_gather(buf, [row_base + (j + 2)])
                    x3 = plsc.load_gather(buf, [row_base + (j + 3)])
                    a0 = a0 + jnp.maximum(x0 + a, 0.0)
                    a1 = a1 + jnp.maximum(x1 + a, 0.0)
                    a2 = a2 + jnp.maximum(x2 + a, 0.0)
                    a3 = a3 + jnp.maximum(x3 + a, 0.0)
                    return (a0, a1, a2, a3)

                a0, a1, a2, a3 = lax.fori_loop(
                    0, C // 4, jbody, (zero, zero, zero, zero)
                )
                acc = (a0 + a1) + (a2 + a3)
                out_v[pl.ds(loc, _LANES)] = (acc - 1.0) * (1.0 / C)

        pltpu.sync_copy(out_v, out_hbm.at[pl.ds(base, R)])

    return sc_hinge


@jax.jit
def kernel(output, y):
    B, C = output.shape
    y32 = y.astype(jnp.int32)
    return _make_sc_hinge(B, C)(output.reshape(B * C), y32)
